# baseline (device time: 688008 ns/iter reference)
import jax
import jax.numpy as jnp
from jax import lax
from jax.experimental import pallas as pl
from jax.experimental.pallas import tpu as pltpu

N_DEV = 16
NZ = 4
KMAX = NZ - 1


def kernel(x):
    m_per, n = x.shape
    half = m_per // 2
    x16 = x.astype(jnp.bfloat16)

    def body(
        x_ref, out_ref, dummy_ref, copy_sem,
        up_s, up_r, dn_s, dn_r,
        fx_own_s, fx_own_r, fy_own_s, fy_own_r,
        fx_up_s, fx_up_r, fx_dn_s, fx_dn_r,
        fy_up_s, fy_up_r, fy_dn_s, fy_dn_r,
        gy_own_s, gy_own_r, hx_own_s, hx_own_r,
        gy_up_s, gy_up_r, gy_dn_s, gy_dn_r,
        hx_up_s, hx_up_r, hx_dn_s, hx_dn_r,
        zx_up_s, zx_up_r, zy_up_s, zy_up_r,
        zx_dn_s, zx_dn_r, zy_dn_s, zy_dn_r,
    ):
        my = lax.axis_index("i")
        z = lax.div(my, 4)
        s = lax.rem(my, 4)
        px = 4 * z + jnp.bitwise_xor(s, 1)
        py = 4 * z + (3 - s)
        up = my + 4
        dn = my - 4

        def full(o):
            return out_ref.at[pl.ds(o * m_per, m_per), :]

        def top(o):
            return out_ref.at[pl.ds(o * m_per, half), :]

        def bot(o):
            return out_ref.at[pl.ds(o * m_per + half, half), :]

        def rdma(src, dst, ssem, rsem, target):
            return pltpu.make_async_remote_copy(
                src_ref=src, dst_ref=dst, send_sem=ssem, recv_sem=rsem,
                device_id=(target,), device_id_type=pl.DeviceIdType.MESH,
            )

        barrier_sem = pltpu.get_barrier_semaphore()
        for nbr in (px, py):
            pl.semaphore_signal(
                barrier_sem, inc=1,
                device_id=(nbr,), device_id_type=pl.DeviceIdType.MESH,
            )

        @pl.when(z < NZ - 1)
        def _():
            pl.semaphore_signal(
                barrier_sem, inc=1,
                device_id=(up,), device_id_type=pl.DeviceIdType.MESH,
            )

        @pl.when(z > 0)
        def _():
            pl.semaphore_signal(
                barrier_sem, inc=1,
                device_id=(dn,), device_id_type=pl.DeviceIdType.MESH,
            )

        deg = 2 + (z < NZ - 1).astype(jnp.int32) + (z > 0).astype(jnp.int32)
        pl.semaphore_wait(barrier_sem, deg)

        local_copy = pltpu.make_async_copy(
            x_ref, full(my), copy_sem
        )
        local_copy.start()

        @pl.when(z < NZ - 1)
        def _():
            rdma(x_ref, full(my), up_s.at[0], up_r.at[0], up).start()

        @pl.when(z > 0)
        def _():
            rdma(x_ref, full(my), dn_s.at[0], dn_r.at[0], dn).start()

        rdma(x_ref, full(my), fx_own_s, fx_own_r, px).start()
        rdma(x_ref, full(my), fy_own_s, fy_own_r, py).start()

        for k in range(KMAX):
            @pl.when(z >= k + 1)
            def _(k=k):
                o = my - 4 * (k + 1)
                rdma(full(o), full(o), up_s.at[k], up_r.at[k], up).wait_recv()
                if k < KMAX - 1:
                    @pl.when(z < NZ - 1)
                    def _():
                        rdma(
                            full(o), full(o),
                            up_s.at[k + 1], up_r.at[k + 1], up,
                        ).start()
                if k > 0:
                    rdma(full(o), full(o), fx_up_s.at[k], fx_up_r.at[k], px).start()
                    rdma(full(o), full(o), fy_up_s.at[k], fy_up_r.at[k], py).start()

            @pl.when(z <= 2 - k)
            def _(k=k):
                o = my + 4 * (k + 1)
                rdma(full(o), full(o), dn_s.at[k], dn_r.at[k], dn).wait_recv()
                if k < KMAX - 1:
                    @pl.when(z > 0)
                    def _():
                        rdma(
                            full(o), full(o),
                            dn_s.at[k + 1], dn_r.at[k + 1], dn,
                        ).start()
                if k > 0:
                    rdma(full(o), full(o), fx_dn_s.at[k], fx_dn_r.at[k], px).start()
                    rdma(full(o), full(o), fy_dn_s.at[k], fy_dn_r.at[k], py).start()
                else:
                    @pl.when(z > 0)
                    def _():
                        rdma(full(o), full(o), fx_dn_s.at[0], fx_dn_r.at[0], px).start()
                        rdma(full(o), full(o), fy_dn_s.at[0], fy_dn_r.at[0], py).start()

        rdma(full(px), full(px), fx_own_s, fx_own_r, px).wait_recv()
        rdma(top(px), top(px), gy_own_s, gy_own_r, py).start()

        @pl.when(z < NZ - 1)
        def _():
            rdma(full(px), full(px), zx_up_s, zx_up_r, up).start()

        @pl.when(z == 1)
        def _():
            rdma(full(px), full(px), zx_dn_s, zx_dn_r, dn).start()

        rdma(full(py), full(py), fy_own_s, fy_own_r, py).wait_recv()
        rdma(bot(py), bot(py), hx_own_s, hx_own_r, px).start()

        @pl.when(z < NZ - 1)
        def _():
            rdma(full(py), full(py), zy_up_s, zy_up_r, up).start()

        @pl.when(z == 1)
        def _():
            rdma(full(py), full(py), zy_dn_s, zy_dn_r, dn).start()

        @pl.when(z >= 1)
        def _():
            o = px - 4
            rdma(full(o), full(o), zx_up_s, zx_up_r, up).wait_recv()
            rdma(top(o), top(o), gy_up_s.at[0], gy_up_r.at[0], py).start()
            o = py - 4
            rdma(full(o), full(o), zy_up_s, zy_up_r, up).wait_recv()
            rdma(bot(o), bot(o), hx_up_s.at[0], hx_up_r.at[0], px).start()

        @pl.when(z == 0)
        def _():
            o = px + 4
            rdma(full(o), full(o), zx_dn_s, zx_dn_r, dn).wait_recv()
            rdma(top(o), top(o), gy_dn_s.at[0], gy_dn_r.at[0], py).start()
            o = py + 4
            rdma(full(o), full(o), zy_dn_s, zy_dn_r, dn).wait_recv()
            rdma(bot(o), bot(o), hx_dn_s.at[0], hx_dn_r.at[0], px).start()

        for k in range(KMAX):
            if k > 0:
                @pl.when(z >= k + 1)
                def _(k=k):
                    o = px - 4 * (k + 1)
                    rdma(full(o), full(o), fx_up_s.at[k], fx_up_r.at[k], px).wait_recv()
                    rdma(top(o), top(o), gy_up_s.at[k], gy_up_r.at[k], py).start()
                    o = py - 4 * (k + 1)
                    rdma(full(o), full(o), fy_up_s.at[k], fy_up_r.at[k], py).wait_recv()
                    rdma(bot(o), bot(o), hx_up_s.at[k], hx_up_r.at[k], px).start()

            cond = (z <= 2 - k) if k > 0 else jnp.logical_and(z <= 2, z >= 1)

            @pl.when(cond)
            def _(k=k):
                o = px + 4 * (k + 1)
                rdma(full(o), full(o), fx_dn_s.at[k], fx_dn_r.at[k], px).wait_recv()
                rdma(top(o), top(o), gy_dn_s.at[k], gy_dn_r.at[k], py).start()
                o = py + 4 * (k + 1)
                rdma(full(o), full(o), fy_dn_s.at[k], fy_dn_r.at[k], py).wait_recv()
                rdma(bot(o), bot(o), hx_dn_s.at[k], hx_dn_r.at[k], px).start()

        dg = 4 * z + (3 - jnp.bitwise_xor(s, 1))
        rdma(bot(dg), bot(dg), hx_own_s, hx_own_r, px).wait_recv()
        rdma(top(dg), top(dg), gy_own_s, gy_own_r, py).wait_recv()
        for k in range(KMAX):
            @pl.when(z >= k + 1)
            def _(k=k):
                o = dg - 4 * (k + 1)
                rdma(bot(o), bot(o), hx_up_s.at[k], hx_up_r.at[k], px).wait_recv()
                rdma(top(o), top(o), gy_up_s.at[k], gy_up_r.at[k], py).wait_recv()

            @pl.when(z <= 2 - k)
            def _(k=k):
                o = dg + 4 * (k + 1)
                rdma(bot(o), bot(o), hx_dn_s.at[k], hx_dn_r.at[k], px).wait_recv()
                rdma(top(o), top(o), gy_dn_s.at[k], gy_dn_r.at[k], py).wait_recv()

        @pl.when(z < NZ - 1)
        def _():
            rdma(x_ref, full(my), up_s.at[0], up_r.at[0], up).wait_send()

        @pl.when(z > 0)
        def _():
            rdma(x_ref, full(my), dn_s.at[0], dn_r.at[0], dn).wait_send()

        rdma(x_ref, full(my), fx_own_s, fx_own_r, px).wait_send()
        rdma(x_ref, full(my), fy_own_s, fy_own_r, py).wait_send()
        rdma(top(dg), top(dg), gy_own_s, gy_own_r, py).wait_send()
        rdma(bot(dg), bot(dg), hx_own_s, hx_own_r, px).wait_send()

        @pl.when(z < NZ - 1)
        def _():
            rdma(full(px), full(px), zx_up_s, zx_up_r, up).wait_send()
            rdma(full(py), full(py), zy_up_s, zy_up_r, up).wait_send()

        @pl.when(z == 1)
        def _():
            rdma(full(px), full(px), zx_dn_s, zx_dn_r, dn).wait_send()
            rdma(full(py), full(py), zy_dn_s, zy_dn_r, dn).wait_send()

        for k in range(KMAX):
            @pl.when(z >= k + 1)
            def _(k=k):
                o = my - 4 * (k + 1)
                if k < KMAX - 1:
                    @pl.when(z < NZ - 1)
                    def _():
                        rdma(
                            full(o), full(o),
                            up_s.at[k + 1], up_r.at[k + 1], up,
                        ).wait_send()
                if k > 0:
                    rdma(full(o), full(o), fx_up_s.at[k], fx_up_r.at[k], px).wait_send()
                    rdma(full(o), full(o), fy_up_s.at[k], fy_up_r.at[k], py).wait_send()
                rdma(top(o), top(o), gy_up_s.at[k], gy_up_r.at[k], py).wait_send()
                rdma(bot(o), bot(o), hx_up_s.at[k], hx_up_r.at[k], px).wait_send()

            @pl.when(z <= 2 - k)
            def _(k=k):
                o = my + 4 * (k + 1)
                if k < KMAX - 1:
                    @pl.when(z > 0)
                    def _():
                        rdma(
                            full(o), full(o),
                            dn_s.at[k + 1], dn_r.at[k + 1], dn,
                        ).wait_send()
                if k > 0:
                    rdma(full(o), full(o), fx_dn_s.at[k], fx_dn_r.at[k], px).wait_send()
                    rdma(full(o), full(o), fy_dn_s.at[k], fy_dn_r.at[k], py).wait_send()
                else:
                    @pl.when(z > 0)
                    def _():
                        rdma(full(o), full(o), fx_dn_s.at[0], fx_dn_r.at[0], px).wait_send()
                        rdma(full(o), full(o), fy_dn_s.at[0], fy_dn_r.at[0], py).wait_send()
                rdma(top(o), top(o), gy_dn_s.at[k], gy_dn_r.at[k], py).wait_send()
                rdma(bot(o), bot(o), hx_dn_s.at[k], hx_dn_r.at[k], px).wait_send()

        local_copy.wait()

    dma = pltpu.SemaphoreType.DMA
    out, _ = pl.pallas_call(
        body,
        out_shape=[
            jax.ShapeDtypeStruct((N_DEV * m_per, n), jnp.bfloat16),
            jax.ShapeDtypeStruct((8, 128), jnp.bfloat16),
        ],
        in_specs=[pl.BlockSpec(memory_space=pltpu.VMEM)],
        out_specs=[
            pl.BlockSpec(memory_space=pl.ANY),
            pl.BlockSpec(memory_space=pl.ANY),
        ],
        scratch_shapes=[
            dma,
            dma((KMAX,)), dma((KMAX,)), dma((KMAX,)), dma((KMAX,)),
            dma, dma, dma, dma,
            dma((KMAX,)), dma((KMAX,)), dma((KMAX,)), dma((KMAX,)),
            dma((KMAX,)), dma((KMAX,)), dma((KMAX,)), dma((KMAX,)),
            dma, dma, dma, dma,
            dma((KMAX,)), dma((KMAX,)), dma((KMAX,)), dma((KMAX,)),
            dma((KMAX,)), dma((KMAX,)), dma((KMAX,)), dma((KMAX,)),
            dma, dma, dma, dma,
            dma, dma, dma, dma,
        ],
        compiler_params=pltpu.CompilerParams(collective_id=0),
    )(x16)
    return out


# device time: 644690 ns/iter; 1.0672x vs baseline; 1.0672x over previous
import jax
import jax.numpy as jnp
from jax import lax
from jax.experimental import pallas as pl
from jax.experimental.pallas import tpu as pltpu

N_DEV = 16
NZ = 4
KMAX = NZ - 1


def kernel(x):
    m_per, n = x.shape
    half = m_per // 2
    x16 = x.astype(jnp.bfloat16)

    def body(
        x_ref, out_ref, dummy_ref, copy_sem,
        up_s, up_r, dn_s, dn_r,
        fx_own_s, fx_own_r, fy_own_s, fy_own_r,
        fx_up_s, fx_up_r, fx_dn_s, fx_dn_r,
        fy_up_s, fy_up_r, fy_dn_s, fy_dn_r,
        gy_own_s, gy_own_r, hx_own_s, hx_own_r,
        gy_up_s, gy_up_r, gy_dn_s, gy_dn_r,
        hx_up_s, hx_up_r, hx_dn_s, hx_dn_r,
    ):
        my = lax.axis_index("i")
        z = lax.div(my, 4)
        s = lax.rem(my, 4)
        px = 4 * z + jnp.bitwise_xor(s, 1)
        py = 4 * z + (3 - s)
        up = my + 4
        dn = my - 4

        def full(o):
            return out_ref.at[pl.ds(o * m_per, m_per), :]

        def top(o):
            return out_ref.at[pl.ds(o * m_per, half), :]

        def bot(o):
            return out_ref.at[pl.ds(o * m_per + half, half), :]

        def rdma(src, dst, ssem, rsem, target):
            return pltpu.make_async_remote_copy(
                src_ref=src, dst_ref=dst, send_sem=ssem, recv_sem=rsem,
                device_id=(target,), device_id_type=pl.DeviceIdType.MESH,
            )

        barrier_sem = pltpu.get_barrier_semaphore()
        for nbr in (px, py):
            pl.semaphore_signal(
                barrier_sem, inc=1,
                device_id=(nbr,), device_id_type=pl.DeviceIdType.MESH,
            )

        @pl.when(z < NZ - 1)
        def _():
            pl.semaphore_signal(
                barrier_sem, inc=1,
                device_id=(up,), device_id_type=pl.DeviceIdType.MESH,
            )

        @pl.when(z > 0)
        def _():
            pl.semaphore_signal(
                barrier_sem, inc=1,
                device_id=(dn,), device_id_type=pl.DeviceIdType.MESH,
            )

        deg = 2 + (z < NZ - 1).astype(jnp.int32) + (z > 0).astype(jnp.int32)
        pl.semaphore_wait(barrier_sem, deg)

        local_copy = pltpu.make_async_copy(
            x_ref, full(my), copy_sem
        )
        local_copy.start()

        @pl.when(z < NZ - 1)
        def _():
            rdma(x_ref, full(my), up_s.at[0], up_r.at[0], up).start()

        @pl.when(z > 0)
        def _():
            rdma(x_ref, full(my), dn_s.at[0], dn_r.at[0], dn).start()

        rdma(x_ref, full(my), fx_own_s, fx_own_r, px).start()
        rdma(x_ref, full(my), fy_own_s, fy_own_r, py).start()

        for k in range(KMAX):
            @pl.when(z >= k + 1)
            def _(k=k):
                o = my - 4 * (k + 1)
                rdma(full(o), full(o), up_s.at[k], up_r.at[k], up).wait_recv()
                if k < KMAX - 1:
                    @pl.when(z < NZ - 1)
                    def _():
                        rdma(
                            full(o), full(o),
                            up_s.at[k + 1], up_r.at[k + 1], up,
                        ).start()
                rdma(full(o), full(o), fx_up_s.at[k], fx_up_r.at[k], px).start()
                rdma(full(o), full(o), fy_up_s.at[k], fy_up_r.at[k], py).start()

            @pl.when(z <= 2 - k)
            def _(k=k):
                o = my + 4 * (k + 1)
                rdma(full(o), full(o), dn_s.at[k], dn_r.at[k], dn).wait_recv()
                if k < KMAX - 1:
                    @pl.when(z > 0)
                    def _():
                        rdma(
                            full(o), full(o),
                            dn_s.at[k + 1], dn_r.at[k + 1], dn,
                        ).start()
                rdma(full(o), full(o), fx_dn_s.at[k], fx_dn_r.at[k], px).start()
                rdma(full(o), full(o), fy_dn_s.at[k], fy_dn_r.at[k], py).start()

        rdma(full(px), full(px), fx_own_s, fx_own_r, px).wait_recv()
        rdma(top(px), top(px), gy_own_s, gy_own_r, py).start()
        rdma(full(py), full(py), fy_own_s, fy_own_r, py).wait_recv()
        rdma(bot(py), bot(py), hx_own_s, hx_own_r, px).start()
        for k in range(KMAX):
            @pl.when(z >= k + 1)
            def _(k=k):
                o = px - 4 * (k + 1)
                rdma(full(o), full(o), fx_up_s.at[k], fx_up_r.at[k], px).wait_recv()
                rdma(top(o), top(o), gy_up_s.at[k], gy_up_r.at[k], py).start()
                o = py - 4 * (k + 1)
                rdma(full(o), full(o), fy_up_s.at[k], fy_up_r.at[k], py).wait_recv()
                rdma(bot(o), bot(o), hx_up_s.at[k], hx_up_r.at[k], px).start()

            @pl.when(z <= 2 - k)
            def _(k=k):
                o = px + 4 * (k + 1)
                rdma(full(o), full(o), fx_dn_s.at[k], fx_dn_r.at[k], px).wait_recv()
                rdma(top(o), top(o), gy_dn_s.at[k], gy_dn_r.at[k], py).start()
                o = py + 4 * (k + 1)
                rdma(full(o), full(o), fy_dn_s.at[k], fy_dn_r.at[k], py).wait_recv()
                rdma(bot(o), bot(o), hx_dn_s.at[k], hx_dn_r.at[k], px).start()

        dg = 4 * z + (3 - jnp.bitwise_xor(s, 1))
        rdma(bot(dg), bot(dg), hx_own_s, hx_own_r, px).wait_recv()
        rdma(top(dg), top(dg), gy_own_s, gy_own_r, py).wait_recv()
        for k in range(KMAX):
            @pl.when(z >= k + 1)
            def _(k=k):
                o = dg - 4 * (k + 1)
                rdma(bot(o), bot(o), hx_up_s.at[k], hx_up_r.at[k], px).wait_recv()
                rdma(top(o), top(o), gy_up_s.at[k], gy_up_r.at[k], py).wait_recv()

            @pl.when(z <= 2 - k)
            def _(k=k):
                o = dg + 4 * (k + 1)
                rdma(bot(o), bot(o), hx_dn_s.at[k], hx_dn_r.at[k], px).wait_recv()
                rdma(top(o), top(o), gy_dn_s.at[k], gy_dn_r.at[k], py).wait_recv()

        @pl.when(z < NZ - 1)
        def _():
            rdma(x_ref, full(my), up_s.at[0], up_r.at[0], up).wait_send()

        @pl.when(z > 0)
        def _():
            rdma(x_ref, full(my), dn_s.at[0], dn_r.at[0], dn).wait_send()

        rdma(x_ref, full(my), fx_own_s, fx_own_r, px).wait_send()
        rdma(x_ref, full(my), fy_own_s, fy_own_r, py).wait_send()
        rdma(top(dg), top(dg), gy_own_s, gy_own_r, py).wait_send()
        rdma(bot(dg), bot(dg), hx_own_s, hx_own_r, px).wait_send()
        for k in range(KMAX):
            @pl.when(z >= k + 1)
            def _(k=k):
                o = my - 4 * (k + 1)
                if k < KMAX - 1:
                    @pl.when(z < NZ - 1)
                    def _():
                        rdma(
                            full(o), full(o),
                            up_s.at[k + 1], up_r.at[k + 1], up,
                        ).wait_send()
                rdma(full(o), full(o), fx_up_s.at[k], fx_up_r.at[k], px).wait_send()
                rdma(full(o), full(o), fy_up_s.at[k], fy_up_r.at[k], py).wait_send()
                rdma(top(o), top(o), gy_up_s.at[k], gy_up_r.at[k], py).wait_send()
                rdma(bot(o), bot(o), hx_up_s.at[k], hx_up_r.at[k], px).wait_send()

            @pl.when(z <= 2 - k)
            def _(k=k):
                o = my + 4 * (k + 1)
                if k < KMAX - 1:
                    @pl.when(z > 0)
                    def _():
                        rdma(
                            full(o), full(o),
                            dn_s.at[k + 1], dn_r.at[k + 1], dn,
                        ).wait_send()
                rdma(full(o), full(o), fx_dn_s.at[k], fx_dn_r.at[k], px).wait_send()
                rdma(full(o), full(o), fy_dn_s.at[k], fy_dn_r.at[k], py).wait_send()
                rdma(top(o), top(o), gy_dn_s.at[k], gy_dn_r.at[k], py).wait_send()
                rdma(bot(o), bot(o), hx_dn_s.at[k], hx_dn_r.at[k], px).wait_send()

        local_copy.wait()

    dma = pltpu.SemaphoreType.DMA
    out, _ = pl.pallas_call(
        body,
        out_shape=[
            jax.ShapeDtypeStruct((N_DEV * m_per, n), jnp.bfloat16),
            jax.ShapeDtypeStruct((8, 128), jnp.bfloat16),
        ],
        in_specs=[pl.BlockSpec(memory_space=pltpu.VMEM)],
        out_specs=[
            pl.BlockSpec(memory_space=pl.ANY),
            pl.BlockSpec(memory_space=pl.ANY),
        ],
        scratch_shapes=[
            dma,
            dma((KMAX,)), dma((KMAX,)), dma((KMAX,)), dma((KMAX,)),
            dma, dma, dma, dma,
            dma((KMAX,)), dma((KMAX,)), dma((KMAX,)), dma((KMAX,)),
            dma((KMAX,)), dma((KMAX,)), dma((KMAX,)), dma((KMAX,)),
            dma, dma, dma, dma,
            dma((KMAX,)), dma((KMAX,)), dma((KMAX,)), dma((KMAX,)),
            dma((KMAX,)), dma((KMAX,)), dma((KMAX,)), dma((KMAX,)),
        ],
        compiler_params=pltpu.CompilerParams(collective_id=0),
    )(x16)
    return out


# device time: 644535 ns/iter; 1.0674x vs baseline; 1.0002x over previous
import jax
import jax.numpy as jnp
from jax import lax
from jax.experimental import pallas as pl
from jax.experimental.pallas import tpu as pltpu

N_DEV = 16
NZ = 4
KMAX = NZ - 1


def kernel(x):
    m_per, n = x.shape
    half = m_per // 2
    x16 = x.astype(jnp.bfloat16)

    def body(
        x_ref, out_ref, copy_sem,
        up_s, up_r, dn_s, dn_r,
        fx_own_s, fx_own_r, fy_own_s, fy_own_r,
        fx_up_s, fx_up_r, fx_dn_s, fx_dn_r,
        fy_up_s, fy_up_r, fy_dn_s, fy_dn_r,
        gy_own_s, gy_own_r, hx_own_s, hx_own_r,
        gy_up_s, gy_up_r, gy_dn_s, gy_dn_r,
        hx_up_s, hx_up_r, hx_dn_s, hx_dn_r,
    ):
        my = lax.axis_index("i")
        z = lax.div(my, 4)
        s = lax.rem(my, 4)
        px = 4 * z + jnp.bitwise_xor(s, 1)
        py = 4 * z + (3 - s)
        up = my + 4
        dn = my - 4

        def full(o):
            return out_ref.at[pl.ds(o * m_per, m_per), :]

        def top(o):
            return out_ref.at[pl.ds(o * m_per, half), :]

        def bot(o):
            return out_ref.at[pl.ds(o * m_per + half, half), :]

        def rdma(src, dst, ssem, rsem, target):
            return pltpu.make_async_remote_copy(
                src_ref=src, dst_ref=dst, send_sem=ssem, recv_sem=rsem,
                device_id=(target,), device_id_type=pl.DeviceIdType.MESH,
            )

        barrier_sem = pltpu.get_barrier_semaphore()
        for nbr in (px, py):
            pl.semaphore_signal(
                barrier_sem, inc=1,
                device_id=(nbr,), device_id_type=pl.DeviceIdType.MESH,
            )

        @pl.when(z < NZ - 1)
        def _():
            pl.semaphore_signal(
                barrier_sem, inc=1,
                device_id=(up,), device_id_type=pl.DeviceIdType.MESH,
            )

        @pl.when(z > 0)
        def _():
            pl.semaphore_signal(
                barrier_sem, inc=1,
                device_id=(dn,), device_id_type=pl.DeviceIdType.MESH,
            )

        deg = 2 + (z < NZ - 1).astype(jnp.int32) + (z > 0).astype(jnp.int32)
        pl.semaphore_wait(barrier_sem, deg)

        local_copy = pltpu.make_async_copy(
            x_ref, full(my), copy_sem
        )
        local_copy.start()

        @pl.when(z < NZ - 1)
        def _():
            rdma(x_ref, full(my), up_s.at[0], up_r.at[0], up).start()

        @pl.when(z > 0)
        def _():
            rdma(x_ref, full(my), dn_s.at[0], dn_r.at[0], dn).start()

        rdma(x_ref, full(my), fx_own_s, fx_own_r, px).start()
        rdma(x_ref, full(my), fy_own_s, fy_own_r, py).start()

        for k in range(KMAX):
            @pl.when(z >= k + 1)
            def _(k=k):
                o = my - 4 * (k + 1)
                rdma(full(o), full(o), up_s.at[k], up_r.at[k], up).wait_recv()
                if k < KMAX - 1:
                    @pl.when(z < NZ - 1)
                    def _():
                        rdma(
                            full(o), full(o),
                            up_s.at[k + 1], up_r.at[k + 1], up,
                        ).start()
                rdma(full(o), full(o), fx_up_s.at[k], fx_up_r.at[k], px).start()
                rdma(full(o), full(o), fy_up_s.at[k], fy_up_r.at[k], py).start()

            @pl.when(z <= 2 - k)
            def _(k=k):
                o = my + 4 * (k + 1)
                rdma(full(o), full(o), dn_s.at[k], dn_r.at[k], dn).wait_recv()
                if k < KMAX - 1:
                    @pl.when(z > 0)
                    def _():
                        rdma(
                            full(o), full(o),
                            dn_s.at[k + 1], dn_r.at[k + 1], dn,
                        ).start()
                rdma(full(o), full(o), fx_dn_s.at[k], fx_dn_r.at[k], px).start()
                rdma(full(o), full(o), fy_dn_s.at[k], fy_dn_r.at[k], py).start()

        rdma(full(px), full(px), fx_own_s, fx_own_r, px).wait_recv()
        rdma(top(px), top(px), gy_own_s, gy_own_r, py).start()
        rdma(full(py), full(py), fy_own_s, fy_own_r, py).wait_recv()
        rdma(bot(py), bot(py), hx_own_s, hx_own_r, px).start()
        for k in range(KMAX):
            @pl.when(z >= k + 1)
            def _(k=k):
                o = px - 4 * (k + 1)
                rdma(full(o), full(o), fx_up_s.at[k], fx_up_r.at[k], px).wait_recv()
                rdma(top(o), top(o), gy_up_s.at[k], gy_up_r.at[k], py).start()
                o = py - 4 * (k + 1)
                rdma(full(o), full(o), fy_up_s.at[k], fy_up_r.at[k], py).wait_recv()
                rdma(bot(o), bot(o), hx_up_s.at[k], hx_up_r.at[k], px).start()

            @pl.when(z <= 2 - k)
            def _(k=k):
                o = px + 4 * (k + 1)
                rdma(full(o), full(o), fx_dn_s.at[k], fx_dn_r.at[k], px).wait_recv()
                rdma(top(o), top(o), gy_dn_s.at[k], gy_dn_r.at[k], py).start()
                o = py + 4 * (k + 1)
                rdma(full(o), full(o), fy_dn_s.at[k], fy_dn_r.at[k], py).wait_recv()
                rdma(bot(o), bot(o), hx_dn_s.at[k], hx_dn_r.at[k], px).start()

        dg = 4 * z + (3 - jnp.bitwise_xor(s, 1))
        rdma(bot(dg), bot(dg), hx_own_s, hx_own_r, px).wait_recv()
        rdma(top(dg), top(dg), gy_own_s, gy_own_r, py).wait_recv()
        for k in range(KMAX):
            @pl.when(z >= k + 1)
            def _(k=k):
                o = dg - 4 * (k + 1)
                rdma(bot(o), bot(o), hx_up_s.at[k], hx_up_r.at[k], px).wait_recv()
                rdma(top(o), top(o), gy_up_s.at[k], gy_up_r.at[k], py).wait_recv()

            @pl.when(z <= 2 - k)
            def _(k=k):
                o = dg + 4 * (k + 1)
                rdma(bot(o), bot(o), hx_dn_s.at[k], hx_dn_r.at[k], px).wait_recv()
                rdma(top(o), top(o), gy_dn_s.at[k], gy_dn_r.at[k], py).wait_recv()

        @pl.when(z < NZ - 1)
        def _():
            rdma(x_ref, full(my), up_s.at[0], up_r.at[0], up).wait_send()

        @pl.when(z > 0)
        def _():
            rdma(x_ref, full(my), dn_s.at[0], dn_r.at[0], dn).wait_send()

        rdma(x_ref, full(my), fx_own_s, fx_own_r, px).wait_send()
        rdma(x_ref, full(my), fy_own_s, fy_own_r, py).wait_send()
        rdma(top(dg), top(dg), gy_own_s, gy_own_r, py).wait_send()
        rdma(bot(dg), bot(dg), hx_own_s, hx_own_r, px).wait_send()
        for k in range(KMAX):
            @pl.when(z >= k + 1)
            def _(k=k):
                o = my - 4 * (k + 1)
                if k < KMAX - 1:
                    @pl.when(z < NZ - 1)
                    def _():
                        rdma(
                            full(o), full(o),
                            up_s.at[k + 1], up_r.at[k + 1], up,
                        ).wait_send()
                rdma(full(o), full(o), fx_up_s.at[k], fx_up_r.at[k], px).wait_send()
                rdma(full(o), full(o), fy_up_s.at[k], fy_up_r.at[k], py).wait_send()
                rdma(top(o), top(o), gy_up_s.at[k], gy_up_r.at[k], py).wait_send()
                rdma(bot(o), bot(o), hx_up_s.at[k], hx_up_r.at[k], px).wait_send()

            @pl.when(z <= 2 - k)
            def _(k=k):
                o = my + 4 * (k + 1)
                if k < KMAX - 1:
                    @pl.when(z > 0)
                    def _():
                        rdma(
                            full(o), full(o),
                            dn_s.at[k + 1], dn_r.at[k + 1], dn,
                        ).wait_send()
                rdma(full(o), full(o), fx_dn_s.at[k], fx_dn_r.at[k], px).wait_send()
                rdma(full(o), full(o), fy_dn_s.at[k], fy_dn_r.at[k], py).wait_send()
                rdma(top(o), top(o), gy_dn_s.at[k], gy_dn_r.at[k], py).wait_send()
                rdma(bot(o), bot(o), hx_dn_s.at[k], hx_dn_r.at[k], px).wait_send()

        local_copy.wait()

    dma = pltpu.SemaphoreType.DMA
    out = pl.pallas_call(
        body,
        out_shape=jax.ShapeDtypeStruct((N_DEV * m_per, n), jnp.bfloat16),
        in_specs=[pl.BlockSpec(memory_space=pltpu.VMEM)],
        out_specs=pl.BlockSpec(memory_space=pl.ANY),
        scratch_shapes=[
            dma,
            dma((KMAX,)), dma((KMAX,)), dma((KMAX,)), dma((KMAX,)),
            dma, dma, dma, dma,
            dma((KMAX,)), dma((KMAX,)), dma((KMAX,)), dma((KMAX,)),
            dma((KMAX,)), dma((KMAX,)), dma((KMAX,)), dma((KMAX,)),
            dma, dma, dma, dma,
            dma((KMAX,)), dma((KMAX,)), dma((KMAX,)), dma((KMAX,)),
            dma((KMAX,)), dma((KMAX,)), dma((KMAX,)), dma((KMAX,)),
        ],
        compiler_params=pltpu.CompilerParams(collective_id=0),
    )(x16)
    return out
